# Initial kernel scaffold; baseline (speedup 1.0000x reference)
import jax
import jax.numpy as jnp
from jax.experimental import pallas as pl


def kernel(x, relative_bias_weight):
    raise NotImplementedError("write your pallas kernel here")


# speedup vs baseline: 42.8471x; 42.8471x over previous
import functools

import jax
import jax.numpy as jnp
from jax import lax
from jax.experimental import pallas as pl
from jax.experimental.pallas import tpu as pltpu
from jax.experimental.pallas import tpu_sc as plsc

NUM_HEADS = 16
SEQ = 2048
MAX_DISTANCE = 128
NBIAS = 2 * MAX_DISTANCE + 1
VSTRIDE = 4608
ROWS_PER_WORKER = SEQ // 2
FIRE = 16
WCOL = 416
WT_PAD = 384
BAND_LO_CHUNK = 1904 // 16
BAND_HI_CHUNK = 2176 // 16
NUM_BAND_CHUNKS = BAND_HI_CHUNK - BAND_LO_CHUNK

_PERM_DNUMS = lax.GatherDimensionNumbers(
    offset_dims=(), collapsed_slice_dims=(0,), start_index_map=(0,))


def _lane_perm(vec, idx):
    return lax.gather(vec, idx[:, None], _PERM_DNUMS, (1,),
                      mode=lax.GatherScatterMode.PROMISE_IN_BOUNDS)


@functools.partial(
    pl.kernel,
    out_type=jax.ShapeDtypeStruct((NUM_HEADS * SEQ * SEQ,), jnp.float32),
    mesh=plsc.VectorSubcoreMesh(core_axis_name="c", subcore_axis_name="s"),
    scratch_types=[
        pltpu.VMEM((WCOL,), jnp.float32),
        pltpu.VMEM((8 * VSTRIDE,), jnp.float32),
        pltpu.SemaphoreType.DMA,
    ],
)
def _bias_kernel(wt_hbm, out_hbm, wcol, vbuf, sem):
    core = lax.axis_index("c")
    sub = lax.axis_index("s")
    h = sub
    row_base = core * ROWS_PER_WORKER

    pltpu.sync_copy(wt_hbm.at[pl.ds(pl.multiple_of(h * WT_PAD, 8), WT_PAD)],
                    wcol.at[pl.ds(16, WT_PAD)])

    lanes = lax.iota(jnp.int32, 16)
    zeros = lanes * 0
    lo_vec = _lane_perm(wcol[pl.ds(16, 16)], zeros)
    hi_vec = _lane_perm(wcol[pl.ds(272, 16)], zeros)
    wcol[pl.ds(0, 16)] = lo_vec
    wcol[pl.ds(272, 16)] = hi_vec
    wcol[pl.ds(288, 16)] = hi_vec

    for k in range(8):
        base_k = k * VSTRIDE
        shift = k + 1
        rot = (lanes + shift) & 15
        first = lanes < (16 - shift)

        def fill_lo(c, carry):
            vbuf[pl.ds(pl.multiple_of(base_k + c * 16, 16), 16)] = lo_vec
            return carry

        def fill_band(c, carry):
            a = wcol[pl.ds(pl.multiple_of(c * 16, 16), 16)]
            b = wcol[pl.ds(pl.multiple_of(c * 16 + 16, 16), 16)]
            rot_a = _lane_perm(a, rot)
            rot_b = _lane_perm(b, rot)
            vals = jnp.where(first, rot_a, rot_b)
            vbuf[pl.ds(pl.multiple_of(base_k + 1904 + c * 16, 16), 16)] = vals
            return carry

        def fill_hi(c, carry):
            vbuf[pl.ds(pl.multiple_of(base_k + c * 16, 16), 16)] = hi_vec
            return carry

        lax.fori_loop(0, BAND_LO_CHUNK, fill_lo, 0)
        lax.fori_loop(0, NUM_BAND_CHUNKS, fill_band, 0)
        lax.fori_loop(BAND_HI_CHUNK, VSTRIDE // 16, fill_hi, 0)

    def rows(g, carry):
        i0 = row_base + g * FIRE
        copies = []
        for j in range(FIRE):
            i = i0 + j
            s = (SEQ - 1) - i
            k = lax.bitwise_and(s, 7)
            off = pl.multiple_of(k * VSTRIDE + (s - k), 8)
            dst = pl.multiple_of((h * SEQ + i) * SEQ, 8)
            copies.append(
                pltpu.async_copy(vbuf.at[pl.ds(off, SEQ)],
                                 out_hbm.at[pl.ds(dst, SEQ)], sem)
            )
        for cp in copies:
            cp.wait()
        return carry

    lax.fori_loop(0, ROWS_PER_WORKER // FIRE, rows, 0)


def kernel(x, relative_bias_weight):
    del x
    wt = jnp.pad(relative_bias_weight.T, ((0, 0), (0, WT_PAD - NBIAS)))
    out = _bias_kernel(wt.reshape(-1))
    return out.reshape(NUM_HEADS, SEQ, SEQ)


# speedup vs baseline: 148.2868x; 3.4608x over previous
import functools

import jax
import jax.numpy as jnp
from jax import lax
from jax.experimental import pallas as pl
from jax.experimental.pallas import tpu as pltpu
from jax.experimental.pallas import tpu_sc as plsc

NUM_HEADS = 16
SEQ = 2048
MAX_DISTANCE = 128
NBIAS = 2 * MAX_DISTANCE + 1
ROWS_PER_WORKER = SEQ // 2
BLOCKS_PER_WORKER = ROWS_PER_WORKER // 8
NTILES = 50
WT_PAD = 384
WCOL = 560

_PERM_DNUMS = lax.GatherDimensionNumbers(
    offset_dims=(), collapsed_slice_dims=(0,), start_index_map=(0,))


def _lane_perm(vec, idx):
    return lax.gather(vec, idx[:, None], _PERM_DNUMS, (1,),
                      mode=lax.GatherScatterMode.PROMISE_IN_BOUNDS)


@functools.partial(
    pl.kernel,
    out_type=jax.ShapeDtypeStruct((NUM_HEADS, SEQ, SEQ), jnp.float32),
    mesh=plsc.VectorSubcoreMesh(core_axis_name="c", subcore_axis_name="s"),
    scratch_types=[
        pltpu.VMEM((WCOL,), jnp.float32),
        pltpu.VMEM((8 * NTILES, 128), jnp.float32),
        pltpu.SemaphoreType.DMA,
    ],
)
def _bias_kernel(wt_hbm, out_hbm, wcol, tiles, sem):
    core = lax.axis_index("c")
    sub = lax.axis_index("s")
    h = sub
    block_base = core * BLOCKS_PER_WORKER

    pltpu.sync_copy(wt_hbm.at[pl.ds(pl.multiple_of(h * WT_PAD, 8), WT_PAD)],
                    wcol.at[pl.ds(144, WT_PAD)])

    lanes = lax.iota(jnp.int32, 16)
    zeros = lanes * 0
    lo_vec = _lane_perm(wcol[pl.ds(144, 16)], zeros)
    hi_vec = _lane_perm(wcol[pl.ds(400, 16)], zeros)
    for p in range(0, 144, 16):
        wcol[pl.ds(p, 16)] = lo_vec
    for p in range(400, WCOL, 16):
        wcol[pl.ds(p, 16)] = hi_vec

    def build_tile(t, carry):
        for j in range(8):
            base = t * 8 + (16 - j)
            sigma = lax.bitwise_and(base, 15)
            a0 = base - sigma
            rot = lax.bitwise_and(lanes + sigma, 15)
            first = lanes < (16 - sigma)
            chunks = [
                _lane_perm(wcol[pl.ds(pl.multiple_of(a0 + 16 * u, 16), 16)], rot)
                for u in range(9)
            ]
            for u in range(8):
                vals = jnp.where(first, chunks[u], chunks[u + 1])
                tiles[t * 8 + j, pl.ds(16 * u, 16)] = vals
        return carry

    lax.fori_loop(0, NTILES, build_tile, 0)

    def blocks(rb, carry):
        r = block_base + rb
        copies = []
        for c in range(16):
            tidx = jnp.clip(32 - r + 16 * c, 0, NTILES - 1)
            src = tiles.at[pl.ds(pl.multiple_of(tidx * 8, 8), 8), :]
            dst = out_hbm.at[h,
                             pl.ds(pl.multiple_of(r * 8, 8), 8),
                             pl.ds(128 * c, 128)]
            copies.append(pltpu.async_copy(src, dst, sem))
        for cp in copies:
            cp.wait()
        return carry

    lax.fori_loop(0, BLOCKS_PER_WORKER, blocks, 0)


def kernel(x, relative_bias_weight):
    del x
    wt = jnp.pad(relative_bias_weight.T, ((0, 0), (0, WT_PAD - NBIAS)))
    return _bias_kernel(wt.reshape(-1))


# speedup vs baseline: 149.4302x; 1.0077x over previous
import functools

import jax
import jax.numpy as jnp
from jax import lax
from jax.experimental import pallas as pl
from jax.experimental.pallas import tpu as pltpu
from jax.experimental.pallas import tpu_sc as plsc

NUM_HEADS = 16
SEQ = 2048
MAX_DISTANCE = 128
NBIAS = 2 * MAX_DISTANCE + 1
ROWS_PER_WORKER = SEQ // 2
BLOCKS_PER_WORKER = ROWS_PER_WORKER // 8
NTILES = 50
WT_PAD = 384
WCOL = 560

_PERM_DNUMS = lax.GatherDimensionNumbers(
    offset_dims=(), collapsed_slice_dims=(0,), start_index_map=(0,))


def _lane_perm(vec, idx):
    return lax.gather(vec, idx[:, None], _PERM_DNUMS, (1,),
                      mode=lax.GatherScatterMode.PROMISE_IN_BOUNDS)


@functools.partial(
    pl.kernel,
    out_type=jax.ShapeDtypeStruct((NUM_HEADS, SEQ, SEQ), jnp.float32),
    mesh=plsc.VectorSubcoreMesh(core_axis_name="c", subcore_axis_name="s"),
    scratch_types=[
        pltpu.VMEM((WCOL,), jnp.float32),
        pltpu.VMEM((8 * NTILES, 128), jnp.float32),
        pltpu.SemaphoreType.DMA,
    ],
)
def _bias_kernel(wt_hbm, out_hbm, wcol, tiles, sem):
    core = lax.axis_index("c")
    sub = lax.axis_index("s")
    h = sub
    block_base = core * BLOCKS_PER_WORKER

    pltpu.sync_copy(wt_hbm.at[pl.ds(pl.multiple_of(h * WT_PAD, 8), WT_PAD)],
                    wcol.at[pl.ds(144, WT_PAD)])

    lanes = lax.iota(jnp.int32, 16)
    zeros = lanes * 0
    lo_vec = _lane_perm(wcol[pl.ds(144, 16)], zeros)
    hi_vec = _lane_perm(wcol[pl.ds(400, 16)], zeros)
    for p in range(0, 144, 16):
        wcol[pl.ds(p, 16)] = lo_vec
    for p in range(400, WCOL, 16):
        wcol[pl.ds(p, 16)] = hi_vec

    def build_tile(t, carry):
        for j in range(8):
            base = t * 8 + (16 - j)
            sigma = lax.bitwise_and(base, 15)
            a0 = base - sigma
            rot = lax.bitwise_and(lanes + sigma, 15)
            first = lanes < (16 - sigma)
            chunks = [
                _lane_perm(wcol[pl.ds(pl.multiple_of(a0 + 16 * u, 16), 16)], rot)
                for u in range(9)
            ]
            for u in range(8):
                vals = jnp.where(first, chunks[u], chunks[u + 1])
                tiles[t * 8 + j, pl.ds(16 * u, 16)] = vals
        return carry

    lax.fori_loop(0, NTILES, build_tile, 0)

    def issue_block(r):
        copies = []
        for c in range(16):
            tidx = jnp.clip(32 - r + 16 * c, 0, NTILES - 1)
            src = tiles.at[pl.ds(pl.multiple_of(tidx * 8, 8), 8), :]
            dst = out_hbm.at[h,
                             pl.ds(pl.multiple_of(r * 8, 8), 8),
                             pl.ds(128 * c, 128)]
            copies.append(pltpu.async_copy(src, dst, sem))
        return copies

    issue_block(block_base)

    def blocks(rb, carry):
        copies = issue_block(block_base + rb + 1)
        for cp in copies:
            cp.wait()
        return carry

    lax.fori_loop(0, BLOCKS_PER_WORKER - 1, blocks, 0)
    for c in range(16):
        pltpu.make_async_copy(
            tiles.at[pl.ds(0, 8), :],
            out_hbm.at[h, pl.ds(pl.multiple_of(block_base * 8, 8), 8),
                       pl.ds(128 * c, 128)],
            sem,
        ).wait()


def kernel(x, relative_bias_weight):
    del x
    wt = jnp.pad(relative_bias_weight.T, ((0, 0), (0, WT_PAD - NBIAS)))
    return _bias_kernel(wt.reshape(-1))
